# baseline (speedup 1.0000x reference)
import functools

import jax
import jax.numpy as jnp
from jax import lax
from jax.experimental import pallas as pl
from jax.experimental.pallas import tpu as pltpu


def _fused_attn_kernel(x_ref, wqkv_ref, wproj_ref, bproj_ref, o_ref,
                       merged_scr, *, num_heads):
    n, c = x_ref.shape[1], x_ref.shape[2]
    hd = c // num_heads

    x = x_ref[0].astype(jnp.bfloat16)
    qkv_bf = jnp.dot(x, wqkv_ref[...],
                     preferred_element_type=jnp.float32
                     ).astype(jnp.bfloat16)

    ones_col = jnp.ones((n, 1), dtype=jnp.bfloat16)
    for h in range(num_heads):
        qh = qkv_bf[:, h * hd:(h + 1) * hd]
        kh = qkv_bf[:, c + h * hd:c + (h + 1) * hd]
        vh = qkv_bf[:, 2 * c + h * hd:2 * c + (h + 1) * hd]
        s = lax.dot_general(qh, kh, (((1,), (1,)), ((), ())),
                            preferred_element_type=jnp.float32)
        m = jnp.max(s, axis=-1, keepdims=True)
        p = jnp.exp2((s - m).astype(jnp.bfloat16))
        v_aug = jnp.concatenate([vh, ones_col], axis=-1)
        pv = jnp.dot(p, v_aug,
                     preferred_element_type=jnp.float32)
        inv_l = 1.0 / pv[:, hd:hd + 1]
        merged_scr[:, h * hd:(h + 1) * hd] = (
            pv[:, :hd] * inv_l).astype(merged_scr.dtype)

    out = jnp.dot(merged_scr[...], wproj_ref[...],
                  preferred_element_type=jnp.float32)
    o_ref[0] = (out + bproj_ref[...]).astype(o_ref.dtype)


def kernel(x, wqkv, wproj, bproj):
    B, N, C = x.shape
    H = 12
    hd = C // H
    scale = hd ** (-0.5) * 1.4426950408889634

    wqkv_bf = jnp.concatenate([wqkv[:, :C] * scale, wqkv[:, C:]],
                              axis=1).astype(jnp.bfloat16)
    wproj_bf = wproj.astype(jnp.bfloat16)
    bproj2d = bproj.reshape(1, C).astype(jnp.float32)

    return pl.pallas_call(
        functools.partial(_fused_attn_kernel, num_heads=H),
        out_shape=jax.ShapeDtypeStruct((B, N, C), x.dtype),
        grid=(B,),
        in_specs=[
            pl.BlockSpec((1, N, C), lambda b: (b, 0, 0)),
            pl.BlockSpec((C, 3 * C), lambda b: (0, 0)),
            pl.BlockSpec((C, C), lambda b: (0, 0)),
            pl.BlockSpec((1, C), lambda b: (0, 0)),
        ],
        out_specs=pl.BlockSpec((1, N, C), lambda b: (b, 0, 0)),
        scratch_shapes=[
            pltpu.VMEM((N, C), jnp.bfloat16),
        ],
        compiler_params=pltpu.CompilerParams(
            dimension_semantics=("parallel",),
            vmem_limit_bytes=56 * 1024 * 1024),
    )(x, wqkv_bf, wproj_bf, bproj2d)


# speedup vs baseline: 1.0195x; 1.0195x over previous
import functools

import jax
import jax.numpy as jnp
from jax import lax
from jax.experimental import pallas as pl
from jax.experimental.pallas import tpu as pltpu


def _fused_attn_kernel(x_ref, wqkv_ref, wproj_ref, bproj_ref, o_ref,
                       merged_scr, *, num_heads):
    n, c = x_ref.shape[1], x_ref.shape[2]
    hd = c // num_heads

    x = x_ref[0].astype(jnp.bfloat16)
    qkv_bf = jnp.dot(x, wqkv_ref[...],
                     preferred_element_type=jnp.float32
                     ).astype(jnp.bfloat16)

    ones_col = jnp.ones((n, 1), dtype=jnp.bfloat16)
    group = 4
    out = bproj_ref[...].astype(jnp.float32)
    for g in range(num_heads // group):
        for hg in range(group):
            h = g * group + hg
            qh = qkv_bf[:, h * hd:(h + 1) * hd]
            kh = qkv_bf[:, c + h * hd:c + (h + 1) * hd]
            vh = qkv_bf[:, 2 * c + h * hd:2 * c + (h + 1) * hd]
            s = lax.dot_general(qh, kh, (((1,), (1,)), ((), ())),
                                preferred_element_type=jnp.float32)
            m = jnp.max(s, axis=-1, keepdims=True)
            p = jnp.exp((s - m).astype(jnp.bfloat16))
            v_aug = jnp.concatenate([vh, ones_col], axis=-1)
            pv = jnp.dot(p, v_aug,
                         preferred_element_type=jnp.float32)
            inv_l = 1.0 / pv[:, hd:hd + 1]
            merged_scr[:, h * hd:(h + 1) * hd] = (
                pv[:, :hd] * inv_l).astype(merged_scr.dtype)
        gs, ge = g * group * hd, (g + 1) * group * hd
        out = out + jnp.dot(merged_scr[:, gs:ge], wproj_ref[gs:ge, :],
                            preferred_element_type=jnp.float32)
    o_ref[0] = out.astype(o_ref.dtype)


def kernel(x, wqkv, wproj, bproj):
    B, N, C = x.shape
    H = 12
    hd = C // H
    scale = hd ** (-0.5)

    wqkv_bf = jnp.concatenate([wqkv[:, :C] * scale, wqkv[:, C:]],
                              axis=1).astype(jnp.bfloat16)
    wproj_bf = wproj.astype(jnp.bfloat16)
    bproj2d = bproj.reshape(1, C).astype(jnp.float32)

    return pl.pallas_call(
        functools.partial(_fused_attn_kernel, num_heads=H),
        out_shape=jax.ShapeDtypeStruct((B, N, C), x.dtype),
        grid=(B,),
        in_specs=[
            pl.BlockSpec((1, N, C), lambda b: (b, 0, 0)),
            pl.BlockSpec((C, 3 * C), lambda b: (0, 0)),
            pl.BlockSpec((C, C), lambda b: (0, 0)),
            pl.BlockSpec((1, C), lambda b: (0, 0)),
        ],
        out_specs=pl.BlockSpec((1, N, C), lambda b: (b, 0, 0)),
        scratch_shapes=[
            pltpu.VMEM((N, C), jnp.bfloat16),
        ],
        compiler_params=pltpu.CompilerParams(
            dimension_semantics=("parallel",),
            vmem_limit_bytes=56 * 1024 * 1024),
    )(x, wqkv_bf, wproj_bf, bproj2d)
